# Initial kernel scaffold; baseline (speedup 1.0000x reference)
import jax
import jax.numpy as jnp
from jax.experimental import pallas as pl


def kernel(inp, tgt, emb, gate_w_in, gate_b_in, w_moe_in, w1, gate_w_out, gate_b_out, w_moe_out, out_w, out_b):
    raise NotImplementedError("write your pallas kernel here")


# speedup vs baseline: 2.5230x; 2.5230x over previous
import functools

import jax
import jax.numpy as jnp
from jax.experimental import pallas as pl

B, S, F, I, K, E, TOPK, C = 1, 2048, 768, 768, 5, 8, 2, 256
BETA = 0.5
ST = 256
NS = S // ST
EPAD = 128


def _top2_combine(logits):
    lane = jax.lax.broadcasted_iota(jnp.int32, logits.shape, 1)
    masked = jnp.where(lane < E, logits, -1e30)
    i1 = jnp.argmax(masked, axis=1, keepdims=True)
    v1 = jnp.max(masked, axis=1, keepdims=True)
    masked2 = jnp.where(lane == i1, -1e30, masked)
    i2 = jnp.argmax(masked2, axis=1, keepdims=True)
    v2 = jnp.max(masked2, axis=1, keepdims=True)
    s1 = jax.nn.sigmoid(v1 - v2)
    s2 = 1.0 - s1
    onehot1 = (lane == i1).astype(jnp.float32)
    onehot2 = (lane == i2).astype(jnp.float32)
    return onehot1 * s1 + onehot2 * s2


def _moe_in_kernel(inp_ref, emb_hi_ref, gw_ref, gb_ref, w_ref, out_ref):
    col = inp_ref[...]
    lane = jax.lax.broadcasted_iota(jnp.int32, (ST, C), 1)
    onehot = (col == lane).astype(jnp.float32)
    h = jnp.dot(onehot, emb_hi_ref[...], preferred_element_type=jnp.float32)
    logits = jnp.dot(h, gw_ref[...], preferred_element_type=jnp.float32) + gb_ref[...]
    comb = _top2_combine(logits)
    acc = jnp.zeros((ST, I), jnp.float32)
    for e in range(E):
        ye = jax.lax.dot_general(h, w_ref[e], (((1,), (1,)), ((), ())),
                                 preferred_element_type=jnp.float32)
        acc = acc + comb[:, e:e + 1] * ye
    out_ref[...] = jnp.maximum(acc, 0.0)


def _conv_kernel(h1p_ref, wk_ref, gw_ref, gb_ref, out_ref, comb_ref):
    i = pl.program_id(0)
    blk = h1p_ref[pl.ds(i * ST, ST + 8), :]
    acc = jnp.zeros((ST, I), jnp.float32)
    for kk in range(K):
        xs = jax.lax.slice(blk, (4 + kk, 0), (4 + kk + ST, I))
        acc = acc + jax.lax.dot_general(xs, wk_ref[kk], (((1,), (1,)), ((), ())),
                                        preferred_element_type=jnp.float32)
    h2 = jnp.maximum(acc, 0.0)
    out_ref[...] = h2
    logits = jnp.dot(h2, gw_ref[...], preferred_element_type=jnp.float32) + gb_ref[...]
    comb_ref[...] = _top2_combine(logits)


def _moe_out_kernel(h2_ref, comb_ref, w_ref, out_ref):
    h2 = h2_ref[...]
    comb = comb_ref[...]
    acc = jnp.zeros((ST, F), jnp.float32)
    for e in range(E):
        ye = jax.lax.dot_general(h2, w_ref[e], (((1,), (1,)), ((), ())),
                                 preferred_element_type=jnp.float32)
        acc = acc + comb[:, e:e + 1] * ye
    out_ref[...] = acc


def _post_kernel(o_ref, inp_ref, tgt_ref, emb_ref, owt_ref, ob_ref, out_ref):
    i = pl.program_id(0)
    o = o_ref[...]
    d, sc, sh = o[:, :F], o[:, F:2 * F], o[:, 2 * F:]
    r = jax.lax.broadcasted_iota(jnp.int32, (F, F), 0)
    c = jax.lax.broadcasted_iota(jnp.int32, (F, F), 1)
    tri = (r <= c).astype(jnp.float32)
    cum = jnp.dot(d, tri, preferred_element_type=jnp.float32)
    pos = (i * ST + jax.lax.broadcasted_iota(jnp.int32, (ST, 1), 0)).astype(jnp.float32)
    y = cum / (pos + 1.0) * sc + sh
    y = y - jnp.mean(y, axis=1, keepdims=True)
    nrm = jnp.sqrt(jnp.sum(y * y, axis=1, keepdims=True))
    y = y / (nrm * (F ** -0.5) + 1e-5)
    col = inp_ref[...]
    lane = jax.lax.broadcasted_iota(jnp.int32, (ST, C), 1)
    onehot = (col == lane).astype(jnp.float32)
    x = jnp.dot(onehot, emb_ref[...], preferred_element_type=jnp.float32)
    x0, x1 = x[:, :F], x[:, F:]
    y1 = x0 * BETA + y * (1.0 - BETA)
    y2 = x1 + y1
    cat = jnp.concatenate([y1, y2], axis=1)
    logits = jnp.dot(cat, owt_ref[...], preferred_element_type=jnp.float32) + ob_ref[...]
    m = jnp.max(logits, axis=1, keepdims=True)
    lse = m + jnp.log(jnp.sum(jnp.exp(logits - m), axis=1, keepdims=True))
    tcol = tgt_ref[...]
    tlane = jax.lax.broadcasted_iota(jnp.int32, (ST, C), 1)
    tsel = (tcol == tlane).astype(jnp.float32)
    g = jnp.sum(logits * tsel, axis=1, keepdims=True)
    part = jnp.sum(lse - g, keepdims=True).reshape(1, 1)

    @pl.when(i == 0)
    def _():
        out_ref[...] = jnp.zeros_like(out_ref)

    out_ref[...] += part


def kernel(inp, tgt, emb, gate_w_in, gate_b_in, w_moe_in, w1, gate_w_out, gate_b_out, w_moe_out, out_w, out_b):
    f32 = jnp.float32
    inp2 = inp.reshape(S, 1).astype(jnp.int32)
    tgt2 = tgt.reshape(S, 1).astype(jnp.int32)
    emb_hi = emb[:, F:]
    gw_in = jnp.zeros((F, EPAD), f32).at[:, :E].set(gate_w_in)
    gb_in = jnp.zeros((1, EPAD), f32).at[0, :E].set(gate_b_in)
    gw_out = jnp.zeros((I, EPAD), f32).at[:, :E].set(gate_w_out)
    gb_out = jnp.zeros((1, EPAD), f32).at[0, :E].set(gate_b_out)
    wk = jnp.transpose(w1, (2, 0, 1))
    owt = out_w.T
    obr = out_b.reshape(1, C)

    h1 = pl.pallas_call(
        _moe_in_kernel,
        grid=(NS,),
        in_specs=[
            pl.BlockSpec((ST, 1), lambda i: (i, 0)),
            pl.BlockSpec((C, F), lambda i: (0, 0)),
            pl.BlockSpec((F, EPAD), lambda i: (0, 0)),
            pl.BlockSpec((1, EPAD), lambda i: (0, 0)),
            pl.BlockSpec((E, I, F), lambda i: (0, 0, 0)),
        ],
        out_specs=pl.BlockSpec((ST, I), lambda i: (i, 0)),
        out_shape=jax.ShapeDtypeStruct((S, I), f32),
    )(inp2, emb_hi, gw_in, gb_in, w_moe_in)

    h1p = jnp.zeros((S + 8, I), f32).at[8:].set(h1)

    h2, comb2 = pl.pallas_call(
        _conv_kernel,
        grid=(NS,),
        in_specs=[
            pl.BlockSpec((S + 8, I), lambda i: (0, 0)),
            pl.BlockSpec((K, I, I), lambda i: (0, 0, 0)),
            pl.BlockSpec((I, EPAD), lambda i: (0, 0)),
            pl.BlockSpec((1, EPAD), lambda i: (0, 0)),
        ],
        out_specs=[
            pl.BlockSpec((ST, I), lambda i: (i, 0)),
            pl.BlockSpec((ST, EPAD), lambda i: (i, 0)),
        ],
        out_shape=[
            jax.ShapeDtypeStruct((S, I), f32),
            jax.ShapeDtypeStruct((S, EPAD), f32),
        ],
    )(h1p, wk, gw_out, gb_out)

    o_out = pl.pallas_call(
        _moe_out_kernel,
        grid=(3, NS),
        in_specs=[
            pl.BlockSpec((ST, I), lambda o, i: (i, 0)),
            pl.BlockSpec((ST, EPAD), lambda o, i: (i, 0)),
            pl.BlockSpec((E, F, I), lambda o, i: (0, o, 0)),
        ],
        out_specs=pl.BlockSpec((ST, F), lambda o, i: (i, o)),
        out_shape=jax.ShapeDtypeStruct((S, 3 * F), f32),
    )(h2, comb2, w_moe_out)

    tot = pl.pallas_call(
        _post_kernel,
        grid=(NS,),
        in_specs=[
            pl.BlockSpec((ST, 3 * F), lambda i: (i, 0)),
            pl.BlockSpec((ST, 1), lambda i: (i, 0)),
            pl.BlockSpec((ST, 1), lambda i: (i, 0)),
            pl.BlockSpec((C, 2 * F), lambda i: (0, 0)),
            pl.BlockSpec((2 * F, C), lambda i: (0, 0)),
            pl.BlockSpec((1, C), lambda i: (0, 0)),
        ],
        out_specs=pl.BlockSpec((1, 1), lambda i: (0, 0)),
        out_shape=jax.ShapeDtypeStruct((1, 1), f32),
    )(o_out, inp2, tgt2, emb, owt, obr)

    return tot[0, 0] / float(B * S)
